# baseline (speedup 1.0000x reference)
import functools

import jax
import jax.numpy as jnp
from jax import lax
from jax.experimental import pallas as pl
from jax.experimental.pallas import tpu as pltpu
from jax.experimental.pallas import tpu_sc as plsc

VOCAB_SIZE = 1000000
EMB = 64
WORDS = EMB // 2
BATCH = 16384
C = 5

VC = 16384
RB = VC // 4
NBLK = -(-VOCAB_SIZE // VC)
LINES = NBLK * RB
SH_V = VC.bit_length() - 1
SH_M = (VC // 4).bit_length() - 1

NUM_CORES = 2
NUM_SUBCORES = 16
NW = NUM_CORES * NUM_SUBCORES
B_PER_W = BATCH // NW
CB = 128
N_CHUNKS = B_PER_W // CB
NG = CB // 16

_HI_MASK = -65536


def _repack_body(a_ref, b_ref, oa_ref, ob_ref):
    rows = lax.broadcasted_iota(jnp.int32, (EMB, EMB), 0)
    cols = lax.broadcasted_iota(jnp.int32, (EMB, EMB), 1)
    ident = jnp.where(rows == cols, 1.0, 0.0).astype(jnp.bfloat16)
    for src, dst in ((a_ref, oa_ref), (b_ref, ob_ref)):
        xb = src[...].astype(jnp.bfloat16)
        wt = lax.dot_general(
            xb, ident,
            dimension_numbers=(((0,), (0,)), ((), ())),
            preferred_element_type=jnp.float32,
        )
        even = wt[:, 0:32].astype(jnp.bfloat16)
        odd = wt[:, 32:64].astype(jnp.bfloat16)
        lo = lax.convert_element_type(
            lax.bitcast_convert_type(even, jnp.uint16), jnp.uint32
        )
        hi = lax.convert_element_type(
            lax.bitcast_convert_type(odd, jnp.uint16), jnp.uint32
        )
        w = lax.bitcast_convert_type(lo | (hi << 16), jnp.int32)
        dst[...] = jnp.concatenate(
            [w[m * RB:(m + 1) * RB, :] for m in range(4)], axis=1
        )


def _repack(tabT_a, tabT_b):
    return pl.pallas_call(
        _repack_body,
        grid=(NBLK,),
        in_specs=[
            pl.BlockSpec((EMB, VC), lambda i: (0, i)),
            pl.BlockSpec((EMB, VC), lambda i: (0, i)),
        ],
        out_specs=[
            pl.BlockSpec((RB, 128), lambda i: (i, 0)),
            pl.BlockSpec((RB, 128), lambda i: (i, 0)),
        ],
        out_shape=[
            jax.ShapeDtypeStruct((LINES, 128), jnp.int32),
            jax.ShapeDtypeStruct((LINES, 128), jnp.int32),
        ],
    )(tabT_a, tabT_b)


def _body(tgt_hbm, ctx_hbm, ttab_hbm, ctab_hbm, out_hbm,
          idxw_v, idxc_v, hiw_v, hic_v, low_v, loc_v,
          w_rows, c_rows, out_v, sem):
    wid = lax.axis_index("s") * NUM_CORES + lax.axis_index("c")
    base = wid * B_PER_W
    iota = lax.iota(jnp.int32, 16)

    for k in range(N_CHUNKS):
        start = base + k * CB
        pltpu.sync_copy(tgt_hbm.at[pl.ds(start, CB)], idxw_v)
        pltpu.sync_copy(ctx_hbm.at[pl.ds(start * C, CB * C)], idxc_v)

        def split_w(i, carry):
            x = idxw_v[pl.ds(i * 16, 16)]
            hiw_v[pl.ds(i * 16, 16)] = ((x >> SH_V) * RB) + (x & (RB - 1))
            low_v[pl.ds(i * 16, 16)] = ((x >> SH_M) & 3) * WORDS
            return carry

        lax.fori_loop(0, CB // 16, split_w, 0)

        def split_c(i, carry):
            x = idxc_v[pl.ds(i * 16, 16)]
            hic_v[pl.ds(i * 16, 16)] = ((x >> SH_V) * RB) + (x & (RB - 1))
            loc_v[pl.ds(i * 16, 16)] = ((x >> SH_M) & 3) * WORDS
            return carry

        lax.fori_loop(0, CB * C // 16, split_c, 0)

        g1 = pltpu.async_copy(ttab_hbm.at[hiw_v], w_rows, sem)
        g2 = pltpu.async_copy(ctab_hbm.at[hic_v], c_rows, sem)
        g1.wait()
        g2.wait()

        def gbody(g, carry):
            rw = g * 16 + iota
            colw0 = plsc.load_gather(low_v, [rw])
            rc = [rw * C + c for c in range(C)]
            colc0 = [plsc.load_gather(loc_v, [rc[c]]) for c in range(C)]

            def jbody(j, accs):
                w_word = plsc.load_gather(w_rows, [rw, colw0 + j])
                we = plsc.bitcast(w_word << 16, jnp.float32)
                wo = plsc.bitcast(w_word & _HI_MASK, jnp.float32)
                out = []
                for c in range(C):
                    c_word = plsc.load_gather(c_rows, [rc[c], colc0[c] + j])
                    ce = plsc.bitcast(c_word << 16, jnp.float32)
                    co = plsc.bitcast(c_word & _HI_MASK, jnp.float32)
                    out.append(accs[c] + we * ce + wo * co)
                return tuple(out)

            zeros = jnp.zeros((16,), jnp.float32)
            accs = lax.fori_loop(0, WORDS, jbody, (zeros,) * C, unroll=4)
            for c in range(C):
                plsc.store_scatter(out_v, [rc[c]], accs[c])
            return carry

        lax.fori_loop(0, NG, gbody, 0)
        pltpu.sync_copy(out_v, out_hbm.at[pl.ds(start * C, CB * C)])


def kernel(target, context, target_table, context_table):
    tgt = target.reshape(BATCH).astype(jnp.int32)
    ctx = context.reshape(BATCH * C).astype(jnp.int32)
    ttab, ctab = _repack(target_table.T, context_table.T)

    mesh = plsc.VectorSubcoreMesh(core_axis_name="c", subcore_axis_name="s")
    run = functools.partial(
        pl.kernel,
        mesh=mesh,
        compiler_params=pltpu.CompilerParams(needs_layout_passes=False),
        out_type=jax.ShapeDtypeStruct((BATCH * C,), jnp.float32),
        scratch_types=[
            pltpu.VMEM((CB,), jnp.int32),
            pltpu.VMEM((CB * C,), jnp.int32),
            pltpu.VMEM((CB,), jnp.int32),
            pltpu.VMEM((CB * C,), jnp.int32),
            pltpu.VMEM((CB,), jnp.int32),
            pltpu.VMEM((CB * C,), jnp.int32),
            pltpu.VMEM((CB, 128), jnp.int32),
            pltpu.VMEM((CB * C, 128), jnp.int32),
            pltpu.VMEM((CB * C,), jnp.float32),
            pltpu.SemaphoreType.DMA,
        ],
    )(_body)
    out = run(tgt, ctx, ttab, ctab)
    return out.reshape(BATCH, C)


# speedup vs baseline: 1.0453x; 1.0453x over previous
import functools

import jax
import jax.numpy as jnp
from jax import lax
from jax.experimental import pallas as pl
from jax.experimental.pallas import tpu as pltpu
from jax.experimental.pallas import tpu_sc as plsc

VOCAB_SIZE = 1000000
EMB = 64
WORDS = EMB // 2
BATCH = 16384
C = 5

VC = 16384
RB = VC // 4
NBLK = -(-VOCAB_SIZE // VC)
LINES = NBLK * RB
SH_V = VC.bit_length() - 1
SH_M = (VC // 4).bit_length() - 1

NUM_CORES = 2
NUM_SUBCORES = 16
NW = NUM_CORES * NUM_SUBCORES
B_PER_W = BATCH // NW
CB = 128
N_CHUNKS = B_PER_W // CB
NG = CB // 16

_HI_MASK = -65536


def _repack_body(a_ref, b_ref, oa_ref, ob_ref):
    for src, dst in ((a_ref, oa_ref), (b_ref, ob_ref)):
        x = src[...]
        even = x[0:32, :].astype(jnp.bfloat16)
        odd = x[32:64, :].astype(jnp.bfloat16)
        lo = lax.convert_element_type(
            lax.bitcast_convert_type(even, jnp.uint16), jnp.uint32
        )
        hi = lax.convert_element_type(
            lax.bitcast_convert_type(odd, jnp.uint16), jnp.uint32
        )
        w = lax.bitcast_convert_type(lo | (hi << 16), jnp.int32)
        wt = w.T
        dst[...] = jnp.concatenate(
            [wt[m * RB:(m + 1) * RB, :] for m in range(4)], axis=1
        )


def _repack(tabT_a, tabT_b):
    return pl.pallas_call(
        _repack_body,
        grid=(NBLK,),
        in_specs=[
            pl.BlockSpec((EMB, VC), lambda i: (0, i)),
            pl.BlockSpec((EMB, VC), lambda i: (0, i)),
        ],
        out_specs=[
            pl.BlockSpec((RB, 128), lambda i: (i, 0)),
            pl.BlockSpec((RB, 128), lambda i: (i, 0)),
        ],
        out_shape=[
            jax.ShapeDtypeStruct((LINES, 128), jnp.int32),
            jax.ShapeDtypeStruct((LINES, 128), jnp.int32),
        ],
    )(tabT_a, tabT_b)


def _body(tgt_hbm, ctx_hbm, ttab_hbm, ctab_hbm, out_hbm,
          idxw_v, idxc_v, hiw_v, hic_v, low_v, loc_v,
          w_rows, c_rows, out_v, sem):
    wid = lax.axis_index("s") * NUM_CORES + lax.axis_index("c")
    base = wid * B_PER_W
    iota = lax.iota(jnp.int32, 16)

    for k in range(N_CHUNKS):
        start = base + k * CB
        pltpu.sync_copy(tgt_hbm.at[pl.ds(start, CB)], idxw_v)
        pltpu.sync_copy(ctx_hbm.at[pl.ds(start * C, CB * C)], idxc_v)

        def split_w(i, carry):
            x = idxw_v[pl.ds(i * 16, 16)]
            hiw_v[pl.ds(i * 16, 16)] = ((x >> SH_V) * RB) + (x & (RB - 1))
            low_v[pl.ds(i * 16, 16)] = ((x >> SH_M) & 3) * WORDS
            return carry

        lax.fori_loop(0, CB // 16, split_w, 0)

        def split_c(i, carry):
            x = idxc_v[pl.ds(i * 16, 16)]
            hic_v[pl.ds(i * 16, 16)] = ((x >> SH_V) * RB) + (x & (RB - 1))
            loc_v[pl.ds(i * 16, 16)] = ((x >> SH_M) & 3) * WORDS
            return carry

        lax.fori_loop(0, CB * C // 16, split_c, 0)

        g1 = pltpu.async_copy(ttab_hbm.at[hiw_v], w_rows, sem)
        g2 = pltpu.async_copy(ctab_hbm.at[hic_v], c_rows, sem)
        g1.wait()
        g2.wait()

        def gbody(g, carry):
            rw = g * 16 + iota
            colw0 = plsc.load_gather(low_v, [rw])
            rc = [rw * C + c for c in range(C)]
            colc0 = [plsc.load_gather(loc_v, [rc[c]]) for c in range(C)]

            def jbody(j, accs):
                w_word = plsc.load_gather(w_rows, [rw, colw0 + j])
                we = plsc.bitcast(w_word << 16, jnp.float32)
                wo = plsc.bitcast(w_word & _HI_MASK, jnp.float32)
                out = []
                for c in range(C):
                    c_word = plsc.load_gather(c_rows, [rc[c], colc0[c] + j])
                    ce = plsc.bitcast(c_word << 16, jnp.float32)
                    co = plsc.bitcast(c_word & _HI_MASK, jnp.float32)
                    out.append(accs[c] + we * ce + wo * co)
                return tuple(out)

            zeros = jnp.zeros((16,), jnp.float32)
            accs = lax.fori_loop(0, WORDS, jbody, (zeros,) * C, unroll=4)
            for c in range(C):
                plsc.store_scatter(out_v, [iota + (c * CB + g * 16)], accs[c])
            return carry

        lax.fori_loop(0, NG, gbody, 0)
        for c in range(C):
            pltpu.sync_copy(
                out_v.at[pl.ds(c * CB, CB)],
                out_hbm.at[pl.ds(c * BATCH + start, CB)],
            )


def kernel(target, context, target_table, context_table):
    tgt = target.reshape(BATCH).astype(jnp.int32)
    ctx = context.reshape(BATCH * C).astype(jnp.int32)
    ttab, ctab = _repack(target_table.T, context_table.T)

    mesh = plsc.VectorSubcoreMesh(core_axis_name="c", subcore_axis_name="s")
    run = functools.partial(
        pl.kernel,
        mesh=mesh,
        compiler_params=pltpu.CompilerParams(needs_layout_passes=False),
        out_type=jax.ShapeDtypeStruct((BATCH * C,), jnp.float32),
        scratch_types=[
            pltpu.VMEM((CB,), jnp.int32),
            pltpu.VMEM((CB * C,), jnp.int32),
            pltpu.VMEM((CB,), jnp.int32),
            pltpu.VMEM((CB * C,), jnp.int32),
            pltpu.VMEM((CB,), jnp.int32),
            pltpu.VMEM((CB * C,), jnp.int32),
            pltpu.VMEM((CB, 128), jnp.int32),
            pltpu.VMEM((CB * C, 128), jnp.int32),
            pltpu.VMEM((CB * C,), jnp.float32),
            pltpu.SemaphoreType.DMA,
        ],
    )(_body)
    out = run(tgt, ctx, ttab, ctab)
    return out.reshape(C, BATCH).T


# speedup vs baseline: 1.0705x; 1.0241x over previous
import functools

import jax
import jax.numpy as jnp
from jax import lax
from jax.experimental import pallas as pl
from jax.experimental.pallas import tpu as pltpu
from jax.experimental.pallas import tpu_sc as plsc

VOCAB_SIZE = 1000000
EMB = 64
WORDS = EMB // 2
BATCH = 16384
C = 5

VC = 16384
RB = VC // 4
NBLK = -(-VOCAB_SIZE // VC)
LINES = NBLK * RB
SH_V = VC.bit_length() - 1
SH_M = (VC // 4).bit_length() - 1

NUM_CORES = 2
NUM_SUBCORES = 16
NW = NUM_CORES * NUM_SUBCORES
B_PER_W = BATCH // NW
CB = 64
N_CHUNKS = B_PER_W // CB
NG = CB // 16

_HI_MASK = -65536


def _repack_body(a_ref, b_ref, oa_ref, ob_ref):
    for src, dst in ((a_ref, oa_ref), (b_ref, ob_ref)):
        x = src[...]
        even = x[0:32, :].astype(jnp.bfloat16)
        odd = x[32:64, :].astype(jnp.bfloat16)
        lo = lax.convert_element_type(
            lax.bitcast_convert_type(even, jnp.uint16), jnp.uint32
        )
        hi = lax.convert_element_type(
            lax.bitcast_convert_type(odd, jnp.uint16), jnp.uint32
        )
        w = lax.bitcast_convert_type(lo | (hi << 16), jnp.int32)
        wt = w.T
        dst[...] = jnp.concatenate(
            [wt[m * RB:(m + 1) * RB, :] for m in range(4)], axis=1
        )


def _repack(tabT_a, tabT_b):
    return pl.pallas_call(
        _repack_body,
        grid=(NBLK,),
        in_specs=[
            pl.BlockSpec((EMB, VC), lambda i: (0, i)),
            pl.BlockSpec((EMB, VC), lambda i: (0, i)),
        ],
        out_specs=[
            pl.BlockSpec((RB, 128), lambda i: (i, 0)),
            pl.BlockSpec((RB, 128), lambda i: (i, 0)),
        ],
        out_shape=[
            jax.ShapeDtypeStruct((LINES, 128), jnp.int32),
            jax.ShapeDtypeStruct((LINES, 128), jnp.int32),
        ],
    )(tabT_a, tabT_b)


def _body(tgt_hbm, ctx_hbm, ttab_hbm, ctab_hbm, out_hbm,
          idxw_v, idxc_v, hiw_v, hic_v, low_v, loc_v,
          w_rows0, c_rows0, w_rows1, c_rows1, out_v, sem0, sem1):
    wid = lax.axis_index("s") * NUM_CORES + lax.axis_index("c")
    base = wid * B_PER_W
    iota = lax.iota(jnp.int32, 16)
    bufs = ((w_rows0, c_rows0, sem0), (w_rows1, c_rows1, sem1))

    pltpu.sync_copy(tgt_hbm.at[pl.ds(base, B_PER_W)], idxw_v)
    pltpu.sync_copy(ctx_hbm.at[pl.ds(base * C, B_PER_W * C)], idxc_v)

    def split_w(i, carry):
        x = idxw_v[pl.ds(i * 16, 16)]
        hiw_v[pl.ds(i * 16, 16)] = ((x >> SH_V) * RB) + (x & (RB - 1))
        low_v[pl.ds(i * 16, 16)] = ((x >> SH_M) & 3) * WORDS
        return carry

    lax.fori_loop(0, B_PER_W // 16, split_w, 0)

    def split_c(i, carry):
        x = idxc_v[pl.ds(i * 16, 16)]
        hic_v[pl.ds(i * 16, 16)] = ((x >> SH_V) * RB) + (x & (RB - 1))
        loc_v[pl.ds(i * 16, 16)] = ((x >> SH_M) & 3) * WORDS
        return carry

    lax.fori_loop(0, B_PER_W * C // 16, split_c, 0)

    def enqueue(k, slot):
        w_rows, c_rows, sem = bufs[slot]
        g1 = pltpu.async_copy(
            ttab_hbm.at[hiw_v.at[pl.ds(k * CB, CB)]], w_rows, sem
        )
        g2 = pltpu.async_copy(
            ctab_hbm.at[hic_v.at[pl.ds(k * CB * C, CB * C)]], c_rows, sem
        )
        return g1, g2

    handles = {0: enqueue(0, 0), 1: enqueue(1, 1)}

    for k in range(N_CHUNKS):
        slot = k % 2
        w_rows, c_rows, sem = bufs[slot]
        g1, g2 = handles.pop(k)
        g1.wait()
        g2.wait()

        def gbody(g, carry, _k=k, _w=w_rows, _c=c_rows):
            rw = g * 16 + iota
            colw0 = plsc.load_gather(low_v, [_k * CB + rw])
            rc = [rw * C + c for c in range(C)]
            colc0 = [
                plsc.load_gather(loc_v, [_k * CB * C + rc[c]])
                for c in range(C)
            ]

            def jbody(j, accs):
                w_word = plsc.load_gather(_w, [rw, colw0 + j])
                we = plsc.bitcast(w_word << 16, jnp.float32)
                wo = plsc.bitcast(w_word & _HI_MASK, jnp.float32)
                out = []
                for c in range(C):
                    c_word = plsc.load_gather(_c, [rc[c], colc0[c] + j])
                    ce = plsc.bitcast(c_word << 16, jnp.float32)
                    co = plsc.bitcast(c_word & _HI_MASK, jnp.float32)
                    out.append(accs[c] + we * ce + wo * co)
                return tuple(out)

            zeros = jnp.zeros((16,), jnp.float32)
            accs = lax.fori_loop(0, WORDS, jbody, (zeros,) * C, unroll=4)
            for c in range(C):
                plsc.store_scatter(out_v, [iota + (c * CB + g * 16)], accs[c])
            return carry

        lax.fori_loop(0, NG, gbody, 0)
        if k + 2 < N_CHUNKS:
            handles[k + 2] = enqueue(k + 2, slot)
        start = base + k * CB
        for c in range(C):
            pltpu.sync_copy(
                out_v.at[pl.ds(c * CB, CB)],
                out_hbm.at[pl.ds(c * BATCH + start, CB)],
            )


def kernel(target, context, target_table, context_table):
    tgt = target.reshape(BATCH).astype(jnp.int32)
    ctx = context.reshape(BATCH * C).astype(jnp.int32)
    ttab, ctab = _repack(target_table.T, context_table.T)

    mesh = plsc.VectorSubcoreMesh(core_axis_name="c", subcore_axis_name="s")
    run = functools.partial(
        pl.kernel,
        mesh=mesh,
        compiler_params=pltpu.CompilerParams(needs_layout_passes=False),
        out_type=jax.ShapeDtypeStruct((BATCH * C,), jnp.float32),
        scratch_types=[
            pltpu.VMEM((B_PER_W,), jnp.int32),
            pltpu.VMEM((B_PER_W * C,), jnp.int32),
            pltpu.VMEM((B_PER_W,), jnp.int32),
            pltpu.VMEM((B_PER_W * C,), jnp.int32),
            pltpu.VMEM((B_PER_W,), jnp.int32),
            pltpu.VMEM((B_PER_W * C,), jnp.int32),
            pltpu.VMEM((CB, 128), jnp.int32),
            pltpu.VMEM((CB * C, 128), jnp.int32),
            pltpu.VMEM((CB, 128), jnp.int32),
            pltpu.VMEM((CB * C, 128), jnp.int32),
            pltpu.VMEM((CB * C,), jnp.float32),
            pltpu.SemaphoreType.DMA,
            pltpu.SemaphoreType.DMA,
        ],
    )(_body)
    out = run(tgt, ctx, ttab, ctab)
    return out.reshape(C, BATCH).T
